# baseline (speedup 1.0000x reference)
import jax
import jax.numpy as jnp
from jax import lax
from jax.experimental import pallas as pl
from jax.experimental.pallas import tpu as pltpu

MAX_NUMS = 16
NN = 2 * MAX_NUMS
D_IN = 8
HIDDEN = 32
PAIRS = 8
TILE_ROWS = PAIRS * NN

_W0_OFF = 0
_W1_OFF = _W0_OFF + D_IN
_WF0A_OFF = _W1_OFF + HIDDEN
_WF0B_OFF = _WF0A_OFF + HIDDEN
_WF1_OFF = _WF0B_OFF + HIDDEN
_W_ROWS = 112
_B_ROWS = 8

_C = (((0,), (0,)), ((), ()))
_TAB = (((0,), (1,)), ((), ()))


def _body(x_ref, at_ref, maskT_ref, invnT_ref, w_ref, bT_ref,
          bmask_ref, pbig_ref, out_ref, xw_sc, g_sc):
    T = at_ref.shape[0]
    R = T * TILE_ROWS
    TP = T * PAIRS

    w = w_ref[...]
    w0 = w[_W0_OFF:_W0_OFF + D_IN, :]
    w1 = w[_W1_OFF:_W1_OFF + HIDDEN, :]
    wf0a = w[_WF0A_OFF:_WF0A_OFF + HIDDEN, :]
    wf0b = w[_WF0B_OFF:_WF0B_OFF + HIDDEN, :]
    wf1r8 = w[_WF1_OFF:_WF1_OFF + 8, :]
    bT = bT_ref[...]
    b0T = bT[:, 0:1]
    b1T = bT[:, 1:2]
    bf0T = bT[:, 2:3]
    bf1 = bT[0:1, 3:4]
    bmask = bmask_ref[...]

    x_flat = x_ref[...].reshape(R, D_IN)
    xw_sc[...] = lax.dot_general(w0, x_flat, _TAB,
                                 preferred_element_type=jnp.float32
                                 ).astype(jnp.bfloat16)

    def tile_body(t, carry):
        at_rows = jnp.swapaxes(at_ref[t], 1, 2).reshape(TILE_ROWS, NN)
        r2 = jnp.concatenate([at_rows, at_rows], axis=1)
        r4 = jnp.concatenate([r2, r2], axis=1)
        r8 = jnp.concatenate([r4, r4], axis=1)
        bdT = r8 * bmask

        c0 = pl.multiple_of(t * TILE_ROWS, TILE_ROWS)
        agg0 = jnp.dot(xw_sc[:, pl.ds(c0, TILE_ROWS)], bdT,
                       preferred_element_type=jnp.float32)
        h0 = jnp.maximum(agg0 + b0T, 0.0).astype(jnp.bfloat16)
        agg1 = jnp.dot(h0, bdT, preferred_element_type=jnp.float32)
        g_sc[:, pl.ds(c0, TILE_ROWS)] = agg1.astype(jnp.bfloat16)
        return carry

    lax.fori_loop(0, T, tile_body, 0, unroll=16)

    h1 = lax.dot_general(w1, g_sc[...], _C,
                         preferred_element_type=jnp.float32) + b1T
    h1 = jnp.maximum(h1, 0.0)
    hm = (h1 * maskT_ref[0]).astype(jnp.bfloat16)

    pooled = jnp.dot(hm, pbig_ref[...],
                     preferred_element_type=jnp.float32)
    pooled = pooled * invnT_ref[0]

    hg1 = pooled[:, :TP].astype(jnp.bfloat16)
    hg2 = pooled[:, TP:].astype(jnp.bfloat16)
    z = (lax.dot_general(wf0a, hg1, _C, preferred_element_type=jnp.float32)
         + lax.dot_general(wf0b, hg2, _C, preferred_element_type=jnp.float32)
         + bf0T)
    z = jnp.maximum(z, 0.0)
    logit8 = jnp.dot(wf1r8, z.astype(jnp.bfloat16),
                     preferred_element_type=jnp.float32)
    out_ref[...] = jax.nn.sigmoid(logit8 + bf1).reshape(1, 8, TP)


def _forward(x_all, a_cmp, mask, invn, pool_sel, w_slab, b_slab,
             tiles_per_step=16):
    del pool_sel
    num_tiles = x_all.shape[0]
    T = int(tiles_per_step)
    grid = num_tiles // T
    TP = T * PAIRS
    R = T * TILE_ROWS

    bmask = jnp.kron(jnp.eye(PAIRS, dtype=jnp.bfloat16),
                     jnp.ones((NN, NN), jnp.bfloat16))
    bT = b_slab.T

    ridx = jnp.arange(R, dtype=jnp.int32)
    col = ((ridx % NN) // MAX_NUMS) * TP + (ridx // TILE_ROWS) * PAIRS \
        + (ridx % TILE_ROWS) // NN
    pbig = (col[:, None] == jnp.arange(2 * TP, dtype=jnp.int32)[None, :]
            ).astype(jnp.bfloat16)

    maskT = mask.reshape(grid, 1, R)
    invnT = invn.reshape(grid, T, 2, PAIRS).transpose(0, 2, 1, 3) \
        .reshape(grid, 1, 2 * TP)

    def tiled(shape):
        return pl.BlockSpec(shape, lambda i: (i,) + (0,) * (len(shape) - 1))

    def const(shape):
        return pl.BlockSpec(shape, lambda i: (0,) * len(shape))

    out = pl.pallas_call(
        _body,
        out_shape=jax.ShapeDtypeStruct((grid, 8, TP), jnp.float32),
        grid=(grid,),
        in_specs=[
            tiled((T, TILE_ROWS, D_IN)),
            tiled((T, PAIRS, NN, NN)),
            tiled((1, 1, R)),
            tiled((1, 1, 2 * TP)),
            const((_W_ROWS, HIDDEN)),
            const((HIDDEN, _B_ROWS)),
            const((TILE_ROWS, TILE_ROWS)),
            const((R, 2 * TP)),
        ],
        out_specs=pl.BlockSpec((1, 8, TP), lambda i: (i, 0, 0)),
        scratch_shapes=[
            pltpu.VMEM((HIDDEN, R), jnp.bfloat16),
            pltpu.VMEM((HIDDEN, R), jnp.bfloat16),
        ],
        compiler_params=pltpu.CompilerParams(
            dimension_semantics=("parallel",)),
    )(x_all, a_cmp, maskT, invnT, w_slab, bT, bmask, pbig)

    return out[:, 0, :].reshape(-1, 1)


def kernel(x_all, a_cmp, mask, invn, pool_sel, w_slab, b_slab):
    return _forward(x_all, a_cmp, mask, invn, pool_sel, w_slab, b_slab)


# speedup vs baseline: 1.4580x; 1.4580x over previous
import jax
import jax.numpy as jnp
from jax import lax
from jax.experimental import pallas as pl
from jax.experimental.pallas import tpu as pltpu

MAX_NUMS = 16
NN = 2 * MAX_NUMS
D_IN = 8
HIDDEN = 32
PAIRS = 8
TILE_ROWS = PAIRS * NN

_W0_OFF = 0
_W1_OFF = _W0_OFF + D_IN
_WF0A_OFF = _W1_OFF + HIDDEN
_WF0B_OFF = _WF0A_OFF + HIDDEN
_WF1_OFF = _WF0B_OFF + HIDDEN
_W_ROWS = 112
_B_ROWS = 8

_C = (((0,), (0,)), ((), ()))
_TAB = (((0,), (1,)), ((), ()))


def _body(x_ref, at_ref, maskT_ref, invnT_ref, w_ref, bT_ref,
          bmask_ref, pbig_ref, out_ref, xw_sc, g_sc):
    T = at_ref.shape[0]
    R = T * TILE_ROWS
    TP = T * PAIRS

    w = w_ref[...]
    w0 = w[_W0_OFF:_W0_OFF + D_IN, :]
    w1 = w[_W1_OFF:_W1_OFF + HIDDEN, :]
    wf0a = w[_WF0A_OFF:_WF0A_OFF + HIDDEN, :]
    wf0b = w[_WF0B_OFF:_WF0B_OFF + HIDDEN, :]
    wf1r8 = w[_WF1_OFF:_WF1_OFF + 8, :]
    bT = bT_ref[...]
    b0T = bT[:, 0:1]
    b1T = bT[:, 1:2]
    bf0T = bT[:, 2:3]
    bf1 = bT[0:1, 3:4]
    bmask = bmask_ref[...]

    x_flat = x_ref[...].reshape(R, D_IN)
    xw_sc[...] = lax.dot_general(w0, x_flat, _TAB,
                                 preferred_element_type=jnp.float32
                                 ).astype(jnp.bfloat16)

    def tile_body(t, carry):
        at_rows = at_ref[t].reshape(TILE_ROWS, NN)
        r2 = jnp.concatenate([at_rows, at_rows], axis=1)
        r4 = jnp.concatenate([r2, r2], axis=1)
        r8 = jnp.concatenate([r4, r4], axis=1)
        bdT = r8 * bmask

        c0 = pl.multiple_of(t * TILE_ROWS, TILE_ROWS)
        agg0 = jnp.dot(xw_sc[:, pl.ds(c0, TILE_ROWS)], bdT,
                       preferred_element_type=jnp.float32)
        h0 = jnp.maximum(agg0 + b0T, 0.0).astype(jnp.bfloat16)
        agg1 = jnp.dot(h0, bdT, preferred_element_type=jnp.float32)
        g_sc[:, pl.ds(c0, TILE_ROWS)] = agg1.astype(jnp.bfloat16)
        return carry

    lax.fori_loop(0, T, tile_body, 0, unroll=16)

    h1 = lax.dot_general(w1, g_sc[...], _C,
                         preferred_element_type=jnp.float32) + b1T
    h1 = jnp.maximum(h1, 0.0)
    hm = (h1 * maskT_ref[0]).astype(jnp.bfloat16)

    pooled = jnp.dot(hm, pbig_ref[...],
                     preferred_element_type=jnp.float32)
    pooled = pooled * invnT_ref[0]

    hg1 = pooled[:, :TP].astype(jnp.bfloat16)
    hg2 = pooled[:, TP:].astype(jnp.bfloat16)
    z = (lax.dot_general(wf0a, hg1, _C, preferred_element_type=jnp.float32)
         + lax.dot_general(wf0b, hg2, _C, preferred_element_type=jnp.float32)
         + bf0T)
    z = jnp.maximum(z, 0.0)
    logit8 = jnp.dot(wf1r8, z.astype(jnp.bfloat16),
                     preferred_element_type=jnp.float32)
    out_ref[...] = jax.nn.sigmoid(logit8 + bf1).reshape(1, 8, TP)


def _forward(x_all, a_cmp, mask, invn, pool_sel, w_slab, b_slab,
             tiles_per_step=32):
    del pool_sel
    num_tiles = x_all.shape[0]
    T = int(tiles_per_step)
    grid = num_tiles // T
    TP = T * PAIRS
    R = T * TILE_ROWS

    a_t = a_cmp.transpose(0, 1, 3, 2)
    bmask = jnp.kron(jnp.eye(PAIRS, dtype=jnp.bfloat16),
                     jnp.ones((NN, NN), jnp.bfloat16))
    bT = b_slab.T

    ridx = jnp.arange(R, dtype=jnp.int32)
    col = ((ridx % NN) // MAX_NUMS) * TP + (ridx // TILE_ROWS) * PAIRS \
        + (ridx % TILE_ROWS) // NN
    pbig = (col[:, None] == jnp.arange(2 * TP, dtype=jnp.int32)[None, :]
            ).astype(jnp.bfloat16)

    maskT = mask.reshape(grid, 1, R)
    invnT = invn.reshape(grid, T, 2, PAIRS).transpose(0, 2, 1, 3) \
        .reshape(grid, 1, 2 * TP)

    def tiled(shape):
        return pl.BlockSpec(shape, lambda i: (i,) + (0,) * (len(shape) - 1))

    def const(shape):
        return pl.BlockSpec(shape, lambda i: (0,) * len(shape))

    out = pl.pallas_call(
        _body,
        out_shape=jax.ShapeDtypeStruct((grid, 8, TP), jnp.float32),
        grid=(grid,),
        in_specs=[
            tiled((T, TILE_ROWS, D_IN)),
            tiled((T, PAIRS, NN, NN)),
            tiled((1, 1, R)),
            tiled((1, 1, 2 * TP)),
            const((_W_ROWS, HIDDEN)),
            const((HIDDEN, _B_ROWS)),
            const((TILE_ROWS, TILE_ROWS)),
            const((R, 2 * TP)),
        ],
        out_specs=pl.BlockSpec((1, 8, TP), lambda i: (i, 0, 0)),
        scratch_shapes=[
            pltpu.VMEM((HIDDEN, R), jnp.bfloat16),
            pltpu.VMEM((HIDDEN, R), jnp.bfloat16),
        ],
        compiler_params=pltpu.CompilerParams(
            dimension_semantics=("parallel",)),
    )(x_all, a_t, maskT, invnT, w_slab, bT, bmask, pbig)

    return out[:, 0, :].reshape(-1, 1)


def kernel(x_all, a_cmp, mask, invn, pool_sel, w_slab, b_slab):
    return _forward(x_all, a_cmp, mask, invn, pool_sel, w_slab, b_slab)


# speedup vs baseline: 1.4890x; 1.0212x over previous
import jax
import jax.numpy as jnp
from jax import lax
from jax.experimental import pallas as pl
from jax.experimental.pallas import tpu as pltpu

MAX_NUMS = 16
NN = 2 * MAX_NUMS
D_IN = 8
HIDDEN = 32
PAIRS = 8
TILE_ROWS = PAIRS * NN

_W0_OFF = 0
_W1_OFF = _W0_OFF + D_IN
_WF0A_OFF = _W1_OFF + HIDDEN
_WF0B_OFF = _WF0A_OFF + HIDDEN
_WF1_OFF = _WF0B_OFF + HIDDEN
_W_ROWS = 112
_B_ROWS = 8

_C = (((0,), (0,)), ((), ()))
_TAB = (((0,), (1,)), ((), ()))


def _body(x_ref, at_ref, maskT_ref, invnT_ref, w_ref, bT_ref,
          bmask_ref, pbig_ref, out_ref, xw_sc, g_sc):
    T = at_ref.shape[0]
    R = T * TILE_ROWS
    TP = T * PAIRS

    w = w_ref[...]
    w0 = w[_W0_OFF:_W0_OFF + D_IN, :]
    w1 = w[_W1_OFF:_W1_OFF + HIDDEN, :]
    wf0a = w[_WF0A_OFF:_WF0A_OFF + HIDDEN, :]
    wf0b = w[_WF0B_OFF:_WF0B_OFF + HIDDEN, :]
    wf1r8 = w[_WF1_OFF:_WF1_OFF + 8, :]
    bT = bT_ref[...]
    b0T = bT[:, 0:1]
    b1T = bT[:, 1:2]
    bf0T = bT[:, 2:3]
    bf1 = bT[0:1, 3:4]
    bmask = bmask_ref[...]

    x_flat = x_ref[...].reshape(R, D_IN)
    xw_sc[...] = lax.dot_general(w0, x_flat, _TAB,
                                 preferred_element_type=jnp.float32
                                 ).astype(jnp.bfloat16)

    def tile_body(t, carry):
        at_rows = at_ref[t].reshape(TILE_ROWS, NN)
        r2 = jnp.concatenate([at_rows, at_rows], axis=1)
        r4 = jnp.concatenate([r2, r2], axis=1)
        r8 = jnp.concatenate([r4, r4], axis=1)
        bdT = r8 * bmask

        c0 = pl.multiple_of(t * TILE_ROWS, TILE_ROWS)
        agg0 = jnp.dot(xw_sc[:, pl.ds(c0, TILE_ROWS)], bdT,
                       preferred_element_type=jnp.float32)
        h0 = jnp.maximum(agg0 + b0T, 0.0).astype(jnp.bfloat16)
        agg1 = jnp.dot(h0, bdT, preferred_element_type=jnp.float32)
        g_sc[:, pl.ds(c0, TILE_ROWS)] = agg1.astype(jnp.bfloat16)
        return carry

    lax.fori_loop(0, T, tile_body, 0, unroll=16)

    h1 = lax.dot_general(w1, g_sc[...], _C,
                         preferred_element_type=jnp.float32) + b1T
    h1 = jnp.maximum(h1, 0.0)
    hm = (h1 * maskT_ref[0]).astype(jnp.bfloat16)

    pooled = jnp.dot(hm, pbig_ref[...],
                     preferred_element_type=jnp.float32)
    pooled = pooled * invnT_ref[0]

    hg1 = pooled[:, :TP].astype(jnp.bfloat16)
    hg2 = pooled[:, TP:].astype(jnp.bfloat16)
    z = (lax.dot_general(wf0a, hg1, _C, preferred_element_type=jnp.float32)
         + lax.dot_general(wf0b, hg2, _C, preferred_element_type=jnp.float32)
         + bf0T)
    z = jnp.maximum(z, 0.0)
    logit8 = jnp.dot(wf1r8, z.astype(jnp.bfloat16),
                     preferred_element_type=jnp.float32)
    out_ref[...] = jax.nn.sigmoid(logit8 + bf1).reshape(1, 8, TP)


def _forward(x_all, a_cmp, mask, invn, pool_sel, w_slab, b_slab,
             tiles_per_step=16):
    del pool_sel
    num_tiles = x_all.shape[0]
    T = int(tiles_per_step)
    grid = num_tiles // T
    TP = T * PAIRS
    R = T * TILE_ROWS

    a_t = a_cmp.transpose(0, 1, 3, 2)
    bmask = jnp.kron(jnp.eye(PAIRS, dtype=jnp.bfloat16),
                     jnp.ones((NN, NN), jnp.bfloat16))
    bT = b_slab.T

    ridx = jnp.arange(R, dtype=jnp.int32)
    col = ((ridx % NN) // MAX_NUMS) * TP + (ridx // TILE_ROWS) * PAIRS \
        + (ridx % TILE_ROWS) // NN
    pbig = (col[:, None] == jnp.arange(2 * TP, dtype=jnp.int32)[None, :]
            ).astype(jnp.bfloat16)

    maskT = mask.reshape(grid, 1, R)
    invnT = invn.reshape(grid, T, 2, PAIRS).transpose(0, 2, 1, 3) \
        .reshape(grid, 1, 2 * TP)

    def tiled(shape):
        return pl.BlockSpec(shape, lambda i: (i,) + (0,) * (len(shape) - 1))

    def const(shape):
        return pl.BlockSpec(shape, lambda i: (0,) * len(shape))

    out = pl.pallas_call(
        _body,
        out_shape=jax.ShapeDtypeStruct((grid, 8, TP), jnp.float32),
        grid=(grid,),
        in_specs=[
            tiled((T, TILE_ROWS, D_IN)),
            tiled((T, PAIRS, NN, NN)),
            tiled((1, 1, R)),
            tiled((1, 1, 2 * TP)),
            const((_W_ROWS, HIDDEN)),
            const((HIDDEN, _B_ROWS)),
            const((TILE_ROWS, TILE_ROWS)),
            const((R, 2 * TP)),
        ],
        out_specs=pl.BlockSpec((1, 8, TP), lambda i: (i, 0, 0)),
        scratch_shapes=[
            pltpu.VMEM((HIDDEN, R), jnp.bfloat16),
            pltpu.VMEM((HIDDEN, R), jnp.bfloat16),
        ],
        compiler_params=pltpu.CompilerParams(
            dimension_semantics=("parallel",)),
    )(x_all, a_t, maskT, invnT, w_slab, bT, bmask, pbig)

    return out[:, 0, :].reshape(-1, 1)


def kernel(x_all, a_cmp, mask, invn, pool_sel, w_slab, b_slab):
    return _forward(x_all, a_cmp, mask, invn, pool_sel, w_slab, b_slab)
